# baseline (speedup 1.0000x reference)
import jax
import jax.numpy as jnp
from jax import lax
from jax.experimental import pallas as pl
from jax.experimental.pallas import tpu as pltpu
from jax.experimental.pallas import tpu_sc as plsc

NC = 2
NS = 16
NW = NC * NS
LANES = 16
CH = 125
CHC = 125
NB = 2


def _mesh():
    return plsc.VectorSubcoreMesh(core_axis_name="c", subcore_axis_name="s",
                                  num_cores=NC, num_subcores=NS)


def _sc_pass(src, packed, w2d, zeros):
    D = src.shape[1]
    n_pad = zeros.shape[0]
    pk0 = packed[0] if isinstance(packed, (list, tuple)) else packed
    nchunks, K = pk0.shape[1], pk0.shape[2]
    rows_per_tile = nchunks // NS
    n_per_tile = n_pad // NS
    n_full = CH // LANES
    tail = CH - n_full * LANES
    n_outer = rows_per_tile // (2 * NB)
    leftover = rows_per_tile - n_outer * (2 * NB)
    weighted = w2d is not None

    packed2 = packed if isinstance(packed, (list, tuple)) else None

    def body(*refs):
        pk2_h = None
        if weighted:
            (src_h, pk_h, w_h, zero_h, out_h, acc) = refs[:6]
            rest = refs[6:]
        elif packed2 is not None:
            (src_h, pk_h, pk2_h, zero_h, out_h, out2_h, acc) = refs[:7]
            rest = refs[7:]
        else:
            (src_h, pk_h, zero_h, out_h, acc) = refs[:5]
            rest = refs[5:]
        ibufs = rest[:2 * NB]
        rest = rest[2 * NB:]
        if weighted:
            wbufs = rest[:2 * NB]
            rest = rest[2 * NB:]
        rows = rest[:NB]
        sems = rest[NB:]
        isems = sems[:2 * NB]
        gsems = sems[2 * NB:3 * NB]
        ssems = sems[3 * NB:4 * NB]
        cid = lax.axis_index("c")
        sid = lax.axis_index("s")
        row0 = sid * n_per_tile
        base = sid * rows_per_tile

        def zero_acc():
            pltpu.sync_copy(zero_h.at[pl.ds(row0, n_per_tile)],
                            acc.at[pl.ds(row0, n_per_tile)])

        def istart(pkr, s, j):
            pltpu.async_copy(pkr.at[cid, base + j], ibufs[s], isems[s])
            if weighted:
                pltpu.async_copy(w_h.at[cid, base + j], wbufs[s], isems[s])

        def iwait(pkr, s, j):
            pltpu.make_async_copy(pkr.at[cid, base + j], ibufs[s],
                                  isems[s]).wait()
            if weighted:
                pltpu.make_async_copy(w_h.at[cid, base + j], wbufs[s],
                                      isems[s]).wait()

        def gstart(b, s):
            pltpu.async_copy(src_h.at[ibufs[s].at[0]], rows[b], gsems[b])

        def gwait(b, s):
            pltpu.make_async_copy(src_h.at[ibufs[s].at[0]], rows[b],
                                  gsems[b]).wait()

        def sstart(b, s):
            pltpu.async_copy(rows[b], acc.at[ibufs[s].at[1]], ssems[b],
                             add=True)

        def swait(b, s):
            pltpu.make_async_copy(rows[b], acc.at[ibufs[s].at[1]],
                                  ssems[b]).wait()

        def mul_rows(b, s, base_row, ks):
            wv = wbufs[s][pl.ds(base_row, LANES)]
            for k in ks:
                wk = wv[k]
                row = base_row + k
                for t in range(D // LANES):
                    sl = pl.ds(t * LANES, LANES)
                    rows[b][row, sl] = rows[b][row, sl] * wk

        def mul_chunk(b, s):
            def wstep(g, c2):
                mul_rows(b, s, g * LANES, range(LANES))
                return c2
            lax.fori_loop(0, n_full, wstep, 0)
            if tail:
                mul_rows(b, s, CH - LANES, range(LANES - tail, LANES))

        def run_pass(pkr, outr):
            for s in range(4):
                istart(pkr, s, s)
            plsc.subcore_barrier()
            iwait(pkr, 0, 0)
            gstart(0, 0)

            def outer(go, carry):
                j0 = go * 4
                for h in range(2):
                    s0, s1 = 2 * h, 2 * h + 1
                    c0, c1 = j0 + 2 * h, j0 + 2 * h + 1
                    iwait(pkr, s1, c1)
                    gstart(1, s1)
                    gwait(0, s0)
                    if weighted:
                        mul_chunk(0, s0)
                    sstart(0, s0)
                    gwait(1, s1)
                    if weighted:
                        mul_chunk(1, s1)
                    sstart(1, s1)
                    swait(0, s0)

                    @pl.when(go < n_outer - 1)
                    def _r0(s0=s0, c=c0 + 4):
                        istart(pkr, s0, c)
                    ns, nc = (2 * h + 2) % 4, c0 + 2
                    if h == 0:
                        iwait(pkr, ns, nc)
                        gstart(0, ns)
                    else:
                        @pl.when(go < n_outer - 1)
                        def _g0(ns=ns, nc=nc):
                            iwait(pkr, ns, nc)
                            gstart(0, ns)
                    swait(1, s1)

                    @pl.when(go < n_outer - 1)
                    def _r1(s1=s1, c=c1 + 4):
                        istart(pkr, s1, c)
                return carry

            lax.fori_loop(0, n_outer, outer, 0)
            for l in range(n_outer * 4, rows_per_tile):
                istart(pkr, 0, l)
                iwait(pkr, 0, l)
                gstart(0, 0)
                gwait(0, 0)
                if weighted:
                    mul_chunk(0, 0)
                sstart(0, 0)
                swait(0, 0)
            plsc.subcore_barrier()
            pltpu.sync_copy(acc.at[pl.ds(row0, n_per_tile)],
                            outr.at[cid, pl.ds(row0, n_per_tile)])

        zero_acc()
        run_pass(pk_h, out_h)
        if pk2_h is not None:
            zero_acc()
            run_pass(pk2_h, out2_h)

    scratch = [pltpu.VMEM_SHARED((n_pad, D), jnp.float32)]
    scratch += [pltpu.VMEM((K, CH), jnp.int32) for _ in range(2 * NB)]
    if weighted:
        scratch += [pltpu.VMEM((CH,), jnp.float32) for _ in range(2 * NB)]
    scratch += [pltpu.VMEM((CH, D), jnp.float32) for _ in range(NB)]
    scratch += [pltpu.SemaphoreType.DMA for _ in range(4 * NB)]
    one = jax.ShapeDtypeStruct((NC, n_pad, D), jnp.float32)
    dual = isinstance(packed, (list, tuple))
    fn = pl.kernel(body,
                   out_type=[one, one] if dual else one,
                   mesh=_mesh(), scratch_types=scratch)
    if dual:
        return fn(src, packed[0], packed[1], zeros)
    args = (src, packed) + ((w2d,) if weighted else ()) + (zeros,)
    return fn(*args)


def _sc_counts(idx2ds, n_pad):
    nidx = len(idx2ds)
    nchunks = idx2ds[0].shape[0]
    rows_per_tile = nchunks // NW
    cols_per_tile = n_pad // NS

    def body(*refs):
        idx_hs = refs[:nidx]
        out_h = refs[nidx]
        accs = refs[nidx + 1:nidx + 1 + nidx]
        idx_v, ones_v, zbuf = refs[nidx + 1 + nidx:]
        cid = lax.axis_index("c")
        sid = lax.axis_index("s")
        wid = cid * NS + sid
        col0 = sid * cols_per_tile

        def zstep(i, c):
            zbuf[pl.ds(i * LANES, LANES)] = jnp.zeros((LANES,), jnp.float32)
            return c
        lax.fori_loop(0, cols_per_tile // LANES, zstep, 0)
        for t in range(128 // LANES):
            ones_v[pl.ds(t * LANES, LANES)] = jnp.ones((LANES,), jnp.float32)
        for a in range(nidx):
            pltpu.sync_copy(zbuf, accs[a].at[pl.ds(col0, cols_per_tile)])
        plsc.subcore_barrier()

        base = wid * rows_per_tile
        for a in range(nidx):
            pltpu.sync_copy(idx_hs[a].at[pl.ds(base, rows_per_tile)], idx_v)

            def step(j, carry, a=a):
                pltpu.sync_copy(ones_v.at[pl.ds(0, CHC)],
                                accs[a].at[idx_v.at[j]], add=True)
                return carry
            lax.fori_loop(0, rows_per_tile, step, 0)
        plsc.subcore_barrier()
        for a in range(nidx):
            pltpu.sync_copy(accs[a].at[pl.ds(col0, cols_per_tile)],
                            out_h.at[cid, a, pl.ds(col0, cols_per_tile)])

    scratch = ([pltpu.VMEM_SHARED((n_pad,), jnp.float32)] * nidx +
               [pltpu.VMEM((rows_per_tile, CHC), jnp.int32),
                pltpu.VMEM((128,), jnp.float32),
                pltpu.VMEM((cols_per_tile,), jnp.float32)])
    fn = pl.kernel(body,
                   out_type=jax.ShapeDtypeStruct((NC, nidx, n_pad), jnp.float32),
                   mesh=_mesh(), scratch_types=scratch)
    return fn(*idx2ds)


def _tc_inv(cnt_partial):
    nidx, n_pad = cnt_partial.shape[1], cnt_partial.shape[2]

    def body(c_ref, o_ref):
        o_ref[...] = 1.0 / jnp.maximum(c_ref[0] + c_ref[1], 1.0)

    return pl.pallas_call(
        body,
        out_shape=jax.ShapeDtypeStruct((nidx, n_pad), jnp.float32),
    )(cnt_partial)


def _tc_combine_avg(s2, inv2, x2):
    _, n_pad, D = x2.shape
    B = 1024

    def body(s_ref, inv_ref, x_ref, o_ref):
        o_ref[...] = (s_ref[...] * inv_ref[...] + x_ref[...]) * 0.5

    spec = pl.BlockSpec((1, B, D), lambda c, i: (c, i, 0))
    return pl.pallas_call(
        body,
        grid=(2, n_pad // B),
        in_specs=[spec, pl.BlockSpec((1, B, 1), lambda c, i: (c, i, 0)), spec],
        out_specs=spec,
        out_shape=jax.ShapeDtypeStruct((2, n_pad, D), jnp.float32),
    )(s2, inv2, x2)


def _tc_final(p12, p34, inv1, inv2, W1t, b1, W2t, b2, W3t, b3, W4t, b4,
              av):
    n_pad, D = p12.shape[1], p12.shape[2]
    B = 1024

    def body(p12_ref, p34_ref, inv1_ref, inv2_ref,
             W1_ref, b1_ref, W2_ref, b2_ref, W3_ref, b3_ref, W4_ref, b4_ref,
             av_ref, o_ref):
        def head(pre, W_ref, b_ref):
            h = jnp.dot(pre, W_ref[...], preferred_element_type=jnp.float32)
            return jnp.maximum(h + b_ref[...], 0.0)

        a1 = head(p12_ref[0] * inv1_ref[...], W1_ref, b1_ref)
        a2 = head(p12_ref[1] * inv2_ref[...], W2_ref, b2_ref)
        a3 = head(p34_ref[0] * inv1_ref[...], W3_ref, b3_ref)
        a4 = head(p34_ref[1] * inv1_ref[...], W4_ref, b4_ref)
        av = av_ref[...]
        s1 = jnp.sum(a1 * av[0:1, :], axis=1, keepdims=True)
        s2 = jnp.sum(a2 * av[1:2, :], axis=1, keepdims=True)
        s3 = jnp.sum(a3 * av[2:3, :], axis=1, keepdims=True)
        s4 = jnp.sum(a4 * av[3:4, :], axis=1, keepdims=True)
        m = jnp.maximum(jnp.maximum(s1, s2), jnp.maximum(s3, s4))
        e1 = jnp.exp(s1 - m)
        e2 = jnp.exp(s2 - m)
        e3 = jnp.exp(s3 - m)
        e4 = jnp.exp(s4 - m)
        z = e1 + e2 + e3 + e4
        o_ref[...] = (e1 * a1 + e2 * a2 + e3 * a3 + e4 * a4) / z

    pspec = pl.BlockSpec((2, B, D), lambda i: (0, i, 0))
    ispec = pl.BlockSpec((B, 1), lambda i: (i, 0))
    wspec = pl.BlockSpec((D, D), lambda i: (0, 0))
    bspec = pl.BlockSpec((1, D), lambda i: (0, 0))
    return pl.pallas_call(
        body,
        grid=(n_pad // B,),
        in_specs=[pspec, pspec, ispec, ispec,
                  wspec, bspec, wspec, bspec, wspec, bspec, wspec, bspec,
                  pl.BlockSpec((4, D), lambda i: (0, 0))],
        out_specs=pl.BlockSpec((B, D), lambda i: (i, 0)),
        out_shape=jax.ShapeDtypeStruct((n_pad, D), jnp.float32),
    )(p12, p34, inv1, inv2, W1t, b1, W2t, b2, W3t, b3, W4t, b4, av)


def kernel(x_node, x0, x2, x3, edge_index_1, edge_index_2, edge_index_12,
           edge_index_13, edge_weight_1, edge_weight_2, W_s1s, b_s1s,
           W_s2s, b_s2s, W_s121s, b_s121s, W_s131s, b_s131s, att_vec):
    N, D = x_node.shape
    E = edge_index_1.shape[1]
    n_pad = ((N + (NS * LANES) - 1) // (NS * LANES)) * (NS * LANES)

    def chunk_i(v):
        return v.reshape(E // CHC, CHC)

    def chunk_p(v):
        return v.reshape(E // CH, CH)

    def padn(v):
        return jnp.concatenate(
            [v, jnp.zeros((n_pad - N, D), jnp.float32)], axis=0)

    e1s, e1d = chunk_i(edge_index_1[0]), chunk_i(edge_index_1[1])
    e2s, e2d = chunk_i(edge_index_2[0]), chunk_i(edge_index_2[1])
    e12s, e12d = chunk_i(edge_index_12[0]), chunk_i(edge_index_12[1])
    e13s, e13d = chunk_i(edge_index_13[0]), chunk_i(edge_index_13[1])
    w1 = edge_weight_1.reshape(E // CH, CH)
    w2 = edge_weight_2.reshape(E // CH, CH)
    f1s, f1d = chunk_p(edge_index_1[0]), chunk_p(edge_index_1[1])
    f2s, f2d = chunk_p(edge_index_2[0]), chunk_p(edge_index_2[1])
    f12s, f12d = chunk_p(edge_index_12[0]), chunk_p(edge_index_12[1])
    f13s, f13d = chunk_p(edge_index_13[0]), chunk_p(edge_index_13[1])
    P = n_pad

    def pack(g, s):
        return jnp.stack([g, s], axis=1)

    pk_K1 = jnp.stack([pack(f1s, f1d), pack(f2s, f2d)])
    w_K1 = jnp.stack([w1, w2])
    pk_K2 = jnp.stack([pack(f1d, f1s), pack(f2d + P, f2s)])
    pk_K3 = jnp.stack([pack(f12s, f12d), pack(f13s, f13d)])
    pk_K4 = jnp.stack([pack(f12d, f12s), pack(f13d + P, f13s)])
    pk_K5 = jnp.stack([pack(f1d, f1s), pack(f1d + P, f1s)])
    w_K5 = jnp.stack([w1, w1])
    zeros = jnp.zeros((n_pad, D), jnp.float32)
    x0p, x2p, x3p = padn(x0), padn(x2), padn(x3)

    cnt_p = _sc_counts([e1d, e1s, e2d, e2s, e12d, e12s, e13d, e13s], n_pad)
    inv8 = _tc_inv(cnt_p)

    def inv(i):
        return inv8[i].reshape(n_pad, 1)

    s_K1 = _sc_pass(x_node, pk_K1, w_K1, zeros)
    n12 = _tc_combine_avg(s_K1, jnp.stack([inv(0), inv(2)]),
                          jnp.stack([x0p, x2p]))
    s_K2, s_K3 = _sc_pass(n12.reshape(2 * P, D), [pk_K2, pk_K3], None, zeros)
    n23 = _tc_combine_avg(s_K3, jnp.stack([inv(4), inv(6)]),
                          jnp.stack([x2p, x3p]))
    s_K4 = _sc_pass(n23.reshape(2 * P, D), pk_K4, None, zeros)
    n33 = _tc_combine_avg(s_K4, jnp.stack([inv(5), inv(7)]),
                          jnp.stack([x0p, x0p]))
    s_K5 = _sc_pass(n33.reshape(2 * P, D), pk_K5, w_K5, zeros)

    h = _tc_final(s_K2, s_K5, inv(1), inv(3),
                  W_s1s.T, b_s1s.reshape(1, D), W_s2s.T, b_s2s.reshape(1, D),
                  W_s121s.T, b_s121s.reshape(1, D), W_s131s.T,
                  b_s131s.reshape(1, D), att_vec)
    return h[:N]


# speedup vs baseline: 1.0047x; 1.0047x over previous
import jax
import jax.numpy as jnp
from jax import lax
from jax.experimental import pallas as pl
from jax.experimental.pallas import tpu as pltpu
from jax.experimental.pallas import tpu_sc as plsc

NC = 2
NS = 16
NW = NC * NS
LANES = 16
CH = 125
CHC = 125
NB = 2


def _mesh():
    return plsc.VectorSubcoreMesh(core_axis_name="c", subcore_axis_name="s",
                                  num_cores=NC, num_subcores=NS)


def _sc_pass(src, packed, w2d, zeros):
    D = src.shape[1]
    n_pad = zeros.shape[0]
    pk0 = packed[0] if isinstance(packed, (list, tuple)) else packed
    nchunks, K = pk0.shape[1], pk0.shape[2]
    rows_per_tile = nchunks // NS
    n_per_tile = n_pad // NS
    n_full = CH // LANES
    tail = CH - n_full * LANES
    n_outer = rows_per_tile // (2 * NB)
    leftover = rows_per_tile - n_outer * (2 * NB)
    weighted = w2d is not None

    packed2 = packed if isinstance(packed, (list, tuple)) else None

    def body(*refs):
        pk2_h = None
        if weighted:
            (src_h, pk_h, w_h, zero_h, out_h, acc) = refs[:6]
            rest = refs[6:]
        elif packed2 is not None:
            (src_h, pk_h, pk2_h, zero_h, out_h, out2_h, acc) = refs[:7]
            rest = refs[7:]
        else:
            (src_h, pk_h, zero_h, out_h, acc) = refs[:5]
            rest = refs[5:]
        ibufs = rest[:2 * NB]
        rest = rest[2 * NB:]
        if weighted:
            wbufs = rest[:2 * NB]
            rest = rest[2 * NB:]
        rows = rest[:NB]
        sems = rest[NB:]
        isems = sems[:2 * NB]
        gsems = sems[2 * NB:3 * NB]
        ssems = sems[3 * NB:4 * NB]
        cid = lax.axis_index("c")
        sid = lax.axis_index("s")
        row0 = sid * n_per_tile
        base = sid * rows_per_tile

        def zero_acc():
            pltpu.sync_copy(zero_h.at[pl.ds(row0, n_per_tile)],
                            acc.at[pl.ds(row0, n_per_tile)])

        def istart(pkr, s, j):
            pltpu.async_copy(pkr.at[cid, base + j], ibufs[s], isems[s])
            if weighted:
                pltpu.async_copy(w_h.at[cid, base + j], wbufs[s], isems[s])

        def iwait(pkr, s, j):
            pltpu.make_async_copy(pkr.at[cid, base + j], ibufs[s],
                                  isems[s]).wait()
            if weighted:
                pltpu.make_async_copy(w_h.at[cid, base + j], wbufs[s],
                                      isems[s]).wait()

        def gstart(b, s):
            pltpu.async_copy(src_h.at[ibufs[s].at[0]], rows[b], gsems[b])

        def gwait(b, s):
            pltpu.make_async_copy(src_h.at[ibufs[s].at[0]], rows[b],
                                  gsems[b]).wait()

        def sstart(b, s):
            pltpu.async_copy(rows[b], acc.at[ibufs[s].at[1]], ssems[b],
                             add=True)

        def swait(b, s):
            pltpu.make_async_copy(rows[b], acc.at[ibufs[s].at[1]],
                                  ssems[b]).wait()

        def mul_rows(b, s, base_row, ks):
            wv = wbufs[s][pl.ds(base_row, LANES)]
            for k in ks:
                wk = wv[k]
                row = base_row + k
                for t in range(D // LANES):
                    sl = pl.ds(t * LANES, LANES)
                    rows[b][row, sl] = rows[b][row, sl] * wk

        def mul_chunk(b, s):
            def wstep(g, c2):
                mul_rows(b, s, g * LANES, range(LANES))
                return c2
            lax.fori_loop(0, n_full, wstep, 0)
            if tail:
                mul_rows(b, s, CH - LANES, range(LANES - tail, LANES))

        def run_pass(pkr, outr):
            for s in range(4):
                istart(pkr, s, s)
            plsc.subcore_barrier()
            iwait(pkr, 0, 0)
            gstart(0, 0)

            def outer(go, carry):
                j0 = go * 4
                for h in range(2):
                    s0, s1 = 2 * h, 2 * h + 1
                    c0, c1 = j0 + 2 * h, j0 + 2 * h + 1
                    iwait(pkr, s1, c1)
                    gstart(1, s1)
                    gwait(0, s0)
                    if weighted:
                        mul_chunk(0, s0)
                    sstart(0, s0)
                    gwait(1, s1)
                    if weighted:
                        mul_chunk(1, s1)
                    sstart(1, s1)
                    swait(0, s0)

                    @pl.when(go < n_outer - 1)
                    def _r0(s0=s0, c=c0 + 4):
                        istart(pkr, s0, c)
                    ns, nc = (2 * h + 2) % 4, c0 + 2
                    if h == 0:
                        iwait(pkr, ns, nc)
                        gstart(0, ns)
                    else:
                        @pl.when(go < n_outer - 1)
                        def _g0(ns=ns, nc=nc):
                            iwait(pkr, ns, nc)
                            gstart(0, ns)
                    swait(1, s1)

                    @pl.when(go < n_outer - 1)
                    def _r1(s1=s1, c=c1 + 4):
                        istart(pkr, s1, c)
                return carry

            lax.fori_loop(0, n_outer, outer, 0)
            for l in range(n_outer * 4, rows_per_tile):
                istart(pkr, 0, l)
                iwait(pkr, 0, l)
                gstart(0, 0)
                gwait(0, 0)
                if weighted:
                    mul_chunk(0, 0)
                sstart(0, 0)
                swait(0, 0)
            plsc.subcore_barrier()
            pltpu.sync_copy(acc.at[pl.ds(row0, n_per_tile)],
                            outr.at[cid, pl.ds(row0, n_per_tile)])

        zero_acc()
        run_pass(pk_h, out_h)
        if pk2_h is not None:
            zero_acc()
            run_pass(pk2_h, out2_h)

    scratch = [pltpu.VMEM_SHARED((n_pad, D), jnp.float32)]
    scratch += [pltpu.VMEM((K, CH), jnp.int32) for _ in range(2 * NB)]
    if weighted:
        scratch += [pltpu.VMEM((CH,), jnp.float32) for _ in range(2 * NB)]
    scratch += [pltpu.VMEM((CH, D), jnp.float32) for _ in range(NB)]
    scratch += [pltpu.SemaphoreType.DMA for _ in range(4 * NB)]
    one = jax.ShapeDtypeStruct((NC, n_pad, D), jnp.float32)
    dual = isinstance(packed, (list, tuple))
    fn = pl.kernel(body,
                   out_type=[one, one] if dual else one,
                   mesh=_mesh(), scratch_types=scratch)
    if dual:
        return fn(src, packed[0], packed[1], zeros)
    args = (src, packed) + ((w2d,) if weighted else ()) + (zeros,)
    return fn(*args)


def _sc_counts(idx2ds, n_pad):
    nidx = len(idx2ds)
    nchunks = idx2ds[0].shape[0]
    rows_per_tile = nchunks // NW
    cols_per_tile = n_pad // NS

    def body(*refs):
        idx_hs = refs[:nidx]
        out_h = refs[nidx]
        accs = refs[nidx + 1:nidx + 1 + nidx]
        idx_v, ones_v, zbuf = refs[nidx + 1 + nidx:]
        cid = lax.axis_index("c")
        sid = lax.axis_index("s")
        wid = cid * NS + sid
        col0 = sid * cols_per_tile

        def zstep(i, c):
            zbuf[pl.ds(i * LANES, LANES)] = jnp.zeros((LANES,), jnp.float32)
            return c
        lax.fori_loop(0, cols_per_tile // LANES, zstep, 0)
        for t in range(128 // LANES):
            ones_v[pl.ds(t * LANES, LANES)] = jnp.ones((LANES,), jnp.float32)
        for a in range(nidx):
            pltpu.sync_copy(zbuf, accs[a].at[pl.ds(col0, cols_per_tile)])
        plsc.subcore_barrier()

        base = wid * rows_per_tile
        for a in range(nidx):
            pltpu.sync_copy(idx_hs[a].at[pl.ds(base, rows_per_tile)], idx_v)

            def step(j, carry, a=a):
                pltpu.sync_copy(ones_v.at[pl.ds(0, CHC)],
                                accs[a].at[idx_v.at[j]], add=True)
                return carry
            lax.fori_loop(0, rows_per_tile, step, 0)
        plsc.subcore_barrier()
        for a in range(nidx):
            pltpu.sync_copy(accs[a].at[pl.ds(col0, cols_per_tile)],
                            out_h.at[cid, a, pl.ds(col0, cols_per_tile)])

    scratch = ([pltpu.VMEM_SHARED((n_pad,), jnp.float32)] * nidx +
               [pltpu.VMEM((rows_per_tile, CHC), jnp.int32),
                pltpu.VMEM((128,), jnp.float32),
                pltpu.VMEM((cols_per_tile,), jnp.float32)])
    fn = pl.kernel(body,
                   out_type=jax.ShapeDtypeStruct((NC, nidx, n_pad), jnp.float32),
                   mesh=_mesh(), scratch_types=scratch)
    return fn(*idx2ds)


def _tc_inv(cnt_partial):
    nidx, n_pad = cnt_partial.shape[1], cnt_partial.shape[2]

    def body(c_ref, o_ref):
        o_ref[...] = 1.0 / jnp.maximum(c_ref[0] + c_ref[1], 1.0)

    return pl.pallas_call(
        body,
        out_shape=jax.ShapeDtypeStruct((nidx, n_pad), jnp.float32),
    )(cnt_partial)


def _tc_combine_avg(s2, inv2, x2):
    _, n_pad, D = x2.shape
    B = 1024

    def body(s_ref, inv_ref, x_ref, o_ref):
        o_ref[...] = (s_ref[...] * inv_ref[...] + x_ref[...]) * 0.5

    spec = pl.BlockSpec((1, B, D), lambda c, i: (c, i, 0))
    return pl.pallas_call(
        body,
        grid=(2, n_pad // B),
        in_specs=[spec, pl.BlockSpec((1, B, 1), lambda c, i: (c, i, 0)), spec],
        out_specs=spec,
        out_shape=jax.ShapeDtypeStruct((2, n_pad, D), jnp.float32),
    )(s2, inv2, x2)


def _tc_final(p12, p34, inv1, inv2, W1t, b1, W2t, b2, W3t, b3, W4t, b4,
              av):
    n_pad, D = p12.shape[1], p12.shape[2]
    B = 1024

    def body(p12_ref, p34_ref, inv1_ref, inv2_ref,
             W1_ref, b1_ref, W2_ref, b2_ref, W3_ref, b3_ref, W4_ref, b4_ref,
             av_ref, o_ref):
        def head(pre, W_ref, b_ref):
            h = jnp.dot(pre, W_ref[...], preferred_element_type=jnp.float32)
            return jnp.maximum(h + b_ref[...], 0.0)

        a1 = head(p12_ref[0] * inv1_ref[...], W1_ref, b1_ref)
        a2 = head(p12_ref[1] * inv2_ref[...], W2_ref, b2_ref)
        a3 = head(p34_ref[0] * inv1_ref[...], W3_ref, b3_ref)
        a4 = head(p34_ref[1] * inv1_ref[...], W4_ref, b4_ref)
        av = av_ref[...]
        s1 = jnp.sum(a1 * av[0:1, :], axis=1, keepdims=True)
        s2 = jnp.sum(a2 * av[1:2, :], axis=1, keepdims=True)
        s3 = jnp.sum(a3 * av[2:3, :], axis=1, keepdims=True)
        s4 = jnp.sum(a4 * av[3:4, :], axis=1, keepdims=True)
        m = jnp.maximum(jnp.maximum(s1, s2), jnp.maximum(s3, s4))
        e1 = jnp.exp(s1 - m)
        e2 = jnp.exp(s2 - m)
        e3 = jnp.exp(s3 - m)
        e4 = jnp.exp(s4 - m)
        z = e1 + e2 + e3 + e4
        o_ref[...] = (e1 * a1 + e2 * a2 + e3 * a3 + e4 * a4) / z

    pspec = pl.BlockSpec((2, B, D), lambda i: (0, i, 0))
    ispec = pl.BlockSpec((B, 1), lambda i: (i, 0))
    wspec = pl.BlockSpec((D, D), lambda i: (0, 0))
    bspec = pl.BlockSpec((1, D), lambda i: (0, 0))
    return pl.pallas_call(
        body,
        grid=(n_pad // B,),
        in_specs=[pspec, pspec, ispec, ispec,
                  wspec, bspec, wspec, bspec, wspec, bspec, wspec, bspec,
                  pl.BlockSpec((4, D), lambda i: (0, 0))],
        out_specs=pl.BlockSpec((B, D), lambda i: (i, 0)),
        out_shape=jax.ShapeDtypeStruct((n_pad, D), jnp.float32),
    )(p12, p34, inv1, inv2, W1t, b1, W2t, b2, W3t, b3, W4t, b4, av)


def kernel(x_node, x0, x2, x3, edge_index_1, edge_index_2, edge_index_12,
           edge_index_13, edge_weight_1, edge_weight_2, W_s1s, b_s1s,
           W_s2s, b_s2s, W_s121s, b_s121s, W_s131s, b_s131s, att_vec):
    N, D = x_node.shape
    E = edge_index_1.shape[1]
    n_pad = ((N + (NS * LANES) - 1) // (NS * LANES)) * (NS * LANES)

    def chunk_i(v):
        return v.reshape(E // CHC, CHC)

    def chunk_p(v):
        return v.reshape(E // CH, CH)

    def padn(v):
        return jnp.concatenate(
            [v, jnp.zeros((n_pad - N, D), jnp.float32)], axis=0)

    e1s, e1d = chunk_i(edge_index_1[0]), chunk_i(edge_index_1[1])
    e2s, e2d = chunk_i(edge_index_2[0]), chunk_i(edge_index_2[1])
    e12s, e12d = chunk_i(edge_index_12[0]), chunk_i(edge_index_12[1])
    e13s, e13d = chunk_i(edge_index_13[0]), chunk_i(edge_index_13[1])
    w1 = edge_weight_1.reshape(E // CH, CH)
    w2 = edge_weight_2.reshape(E // CH, CH)
    f1s, f1d = chunk_p(edge_index_1[0]), chunk_p(edge_index_1[1])
    f2s, f2d = chunk_p(edge_index_2[0]), chunk_p(edge_index_2[1])
    f12s, f12d = chunk_p(edge_index_12[0]), chunk_p(edge_index_12[1])
    f13s, f13d = chunk_p(edge_index_13[0]), chunk_p(edge_index_13[1])
    P = n_pad

    def pack(g, s):
        return jnp.stack([g, s], axis=1)

    pk_K1 = jnp.stack([pack(f1s, f1d), pack(f2s, f2d)])
    w_K1 = jnp.stack([w1, w2])
    pk_K2 = jnp.stack([pack(f1d, f1s), pack(f2d + P, f2s)])
    pk_K3 = jnp.stack([pack(f12s, f12d), pack(f13s, f13d)])
    pk_K4 = jnp.stack([pack(f12d, f12s), pack(f13d + P, f13s)])
    pk_K5 = jnp.stack([pack(f1d, f1s), pack(f1d + P, f1s)])
    w_K5 = jnp.stack([w1, w1])
    zeros = jnp.zeros((n_pad, D), jnp.float32)
    x0p, x2p, x3p = padn(x0), padn(x2), padn(x3)

    cnt_p = _sc_counts([e1d, e1s, e2d, e2s, e12d, e12s, e13d, e13s], n_pad)
    inv8 = _tc_inv(cnt_p)

    def inv(i):
        return inv8[i].reshape(n_pad, 1)

    s_K1 = _sc_pass(x_node, pk_K1, w_K1, zeros)
    n12 = _tc_combine_avg(s_K1, jnp.stack([inv(0), inv(2)]),
                          jnp.stack([x0p, x2p]))
    s_K2 = _sc_pass(n12.reshape(2 * P, D), pk_K2, None, zeros)
    s_K3 = _sc_pass(n12.reshape(2 * P, D), pk_K3, None, zeros)
    n23 = _tc_combine_avg(s_K3, jnp.stack([inv(4), inv(6)]),
                          jnp.stack([x2p, x3p]))
    s_K4 = _sc_pass(n23.reshape(2 * P, D), pk_K4, None, zeros)
    n33 = _tc_combine_avg(s_K4, jnp.stack([inv(5), inv(7)]),
                          jnp.stack([x0p, x0p]))
    s_K5 = _sc_pass(n33.reshape(2 * P, D), pk_K5, w_K5, zeros)

    h = _tc_final(s_K2, s_K5, inv(1), inv(3),
                  W_s1s.T, b_s1s.reshape(1, D), W_s2s.T, b_s2s.reshape(1, D),
                  W_s121s.T, b_s121s.reshape(1, D), W_s131s.T,
                  b_s131s.reshape(1, D), att_vec)
    return h[:N]
